# Initial kernel scaffold; baseline (speedup 1.0000x reference)
import jax
import jax.numpy as jnp
from jax.experimental import pallas as pl


def kernel(x_proj, edge_indices, edge_scores, cancer_type_id, causal_scores, cancer_table, rw_W1_0, rw_b1_0, rw_W2_0, rw_b2_0, lin1_0, lin2_0, rw_W1_1, rw_b1_1, rw_W2_1, rw_b2_1, lin1_1, lin2_1, rw_W1_2, rw_b1_2, rw_W2_2, rw_b2_2, lin1_2, lin2_2):
    raise NotImplementedError("write your pallas kernel here")


# speedup vs baseline: 2.9201x; 2.9201x over previous
import functools

import jax
import jax.numpy as jnp
from jax import lax
from jax.experimental import pallas as pl
from jax.experimental.pallas import tpu as pltpu
from jax.experimental.pallas import tpu_sc as plsc

_N = 10000
_E = 320000
_D = 128
_HID = 145
_HP = 256
_V = 3
_NC = 2
_NS = 16
_NW = _NC * _NS
_C1 = 64
_C2 = 128
_EPW = 10112
_NCH1 = _EPW // _C1
_NCH2 = _EPW // _C2
_EP = _NW * _EPW
_BR = 1000
_GB = _N // _BR
_ZR = 40
_NZ = _N // _ZR
_NZT = -(-_NZ // _NS)
_EPS = 1e-15


def _tc_norm(x):
    return jnp.clip(jnp.sqrt(jnp.sum(x * x, axis=-1, keepdims=True)), _EPS, None)


def _tc_expmap0(u):
    n = _tc_norm(u)
    return jnp.tanh(n) * u / n


def _tc_logmap0(x):
    n = jnp.clip(_tc_norm(x), _EPS, 1.0 - 1e-5)
    return 0.5 * jnp.log((1.0 + n) / (1.0 - n)) * x / n


def _tc_mobius_add(x, y):
    x2 = jnp.sum(x * x, axis=-1, keepdims=True)
    y2 = jnp.sum(y * y, axis=-1, keepdims=True)
    xy = jnp.sum(x * y, axis=-1, keepdims=True)
    num = (1.0 + 2.0 * xy + y2) * x + (1.0 - x2) * y
    den = jnp.clip(1.0 + 2.0 * xy + x2 * y2, _EPS, None)
    return num / den


def _leaky(x):
    return jnp.where(x >= 0, x, 0.1 * x)


def _tc1_body(x_ref, ca_ref, tab_ref, cid_ref,
              w1a_ref, w1b_ref, w1c_ref, u_ref, t_ref, b1_ref, lin1_ref,
              ps_ref, pd_ref, y1_ref, xh_ref):
    xb = x_ref[...]
    ca = ca_ref[...]
    idx = cid_ref[0]
    onehot = (lax.broadcasted_iota(jnp.int32, (16, 1), 0) == idx
              ).astype(jnp.float32)
    ctx = jnp.sum(onehot * tab_ref[...], axis=0, keepdims=True)
    xh = _tc_expmap0(xb)
    xh_ref[...] = xh
    xtan = _tc_logmap0(xh)
    for v in range(_V):
        ps_ref[v] = (jnp.dot(xb, w1a_ref[v], preferred_element_type=jnp.float32)
                     + ca * u_ref[v])
        addv = (jnp.dot(ctx, w1c_ref[v], preferred_element_type=jnp.float32)
                + b1_ref[v])
        pd_ref[v] = (jnp.dot(xb, w1b_ref[v], preferred_element_type=jnp.float32)
                     + ca * t_ref[v] + addv)
        z = jnp.dot(xtan, lin1_ref[v], preferred_element_type=jnp.float32)
        y1_ref[v] = _tc_logmap0(_tc_expmap0(z))


def _tc2_body(pa0, pb0, pa1, pb1, pa2, pb2, y1_ref, xh_ref, lin2_ref,
              h1_ref, y2_ref):
    xh = xh_ref[...]
    pa = (pa0, pa1, pa2)
    pb = (pb0, pb1, pb2)
    for v in range(_V):
        agg = pa[v][...] + pb[v][...] + y1_ref[v]
        h = _tc_expmap0(_leaky(agg))
        h1 = _tc_mobius_add(h, xh)
        h1_ref[v] = h1
        z = jnp.dot(_tc_logmap0(h1), lin2_ref[v],
                    preferred_element_type=jnp.float32)
        y2_ref[v] = _tc_logmap0(_tc_expmap0(z))


def _tc3_body(pa0, pb0, pa1, pb1, pa2, pb2, y2_ref, h1_ref,
              out_ref, h2_ref):
    pa = (pa0, pa1, pa2)
    pb = (pb0, pb1, pb2)
    for v in range(_V):
        agg = pa[v][...] + pb[v][...] + y2_ref[v]
        h = _tc_expmap0(_leaky(agg))
        h2 = _tc_mobius_add(h, h1_ref[v])
        h2_ref[v] = h2
        out_ref[:, v * _D:(v + 1) * _D] = _tc_logmap0(h2)


def _sc1_body(ps0, ps1, ps2, pd0, pd1, pd2, ya, yb, yc,
              se0, se1, se2, de0, de1, de2, sc0, sc1, sc2,
              w2p0, w2p1, w2p2, b2p0, b2p1, b2p2,
              wo0, wo1, wo2, p0, p1, p2,
              sbuf, dbuf, ybuf, zbuf, srci, dsti, scb, wbuf, w2b, b2b, accsh,
              ss, sd, sy):
    cid = lax.axis_index("c")
    sid = lax.axis_index("s")
    wid = sid * _NC + cid
    zv = jnp.zeros((16,), jnp.float32)

    def _zrow(r, _):
        for k in range(_D // 16):
            zbuf[r, pl.ds(k * 16, 16)] = zv
        return 0
    lax.fori_loop(0, _ZR, _zrow, 0)

    eg = [lax.iota(jnp.int32, 16) + 16 * g for g in range(_C1 // 16)]
    base0 = wid * _EPW
    psv_ = (ps0, ps1, ps2)
    pdv_ = (pd0, pd1, pd2)
    yv_ = (ya, yb, yc)
    sev_ = (se0, se1, se2)
    dev_ = (de0, de1, de2)
    scv_ = (sc0, sc1, sc2)
    w2v_ = (w2p0, w2p1, w2p2)
    b2v_ = (b2p0, b2p1, b2p2)
    wov_ = (wo0, wo1, wo2)
    pv_ = (p0, p1, p2)

    for v in range(_V):
        for tt in range(_NZT):
            ch = sid + _NS * tt
            @pl.when(ch < _NZ)
            def _():
                pltpu.sync_copy(zbuf, accsh.at[pl.ds(ch * _ZR, _ZR)])
        pltpu.sync_copy(w2v_[v], w2b)
        pltpu.sync_copy(b2v_[v], b2b)
        plsc.subcore_barrier()
        b2l = b2b[...]

        def _chunk(i, _, v=v):
            base = base0 + i * _C1
            pltpu.sync_copy(sev_[v].at[pl.ds(base, _C1)], srci)
            pltpu.sync_copy(dev_[v].at[pl.ds(base, _C1)], dsti)
            pltpu.sync_copy(scv_[v].at[pl.ds(base, _C1)], scb)
            cs = pltpu.async_copy(psv_[v].at[srci], sbuf, ss)
            cd = pltpu.async_copy(pdv_[v].at[dsti], dbuf, sd)
            cy = pltpu.async_copy(yv_[v].at[srci], ybuf, sy)
            cs.wait()
            cd.wait()

            def _col(col, accs):
                colv = jnp.full((16,), col, jnp.int32)
                w2c = jnp.full((16,), w2b[pl.ds(col, 16)][0])
                out = []
                for g in range(_C1 // 16):
                    sg = plsc.load_gather(sbuf, [eg[g], colv])
                    dg = plsc.load_gather(dbuf, [eg[g], colv])
                    out.append(accs[g] + jnp.maximum(sg + dg, 0.0) * w2c)
                return tuple(out)
            accs = lax.fori_loop(
                0, _HID, _col,
                tuple(jnp.zeros((16,), jnp.float32) for _ in range(_C1 // 16)))
            for g in range(_C1 // 16):
                t = accs[g] + b2l
                dyn = 1.0 / (1.0 + jnp.exp(-t))
                wbuf[pl.ds(g * 16, 16)] = scb[pl.ds(g * 16, 16)] * dyn
            pltpu.sync_copy(wbuf.at[pl.ds(0, _C1)], wov_[v].at[pl.ds(base, _C1)])
            cy.wait()

            def _srow(c, _):
                wv = jnp.full((16,), wbuf[pl.ds(c, 16)][0])
                for k in range(_D // 16):
                    ybuf[c, pl.ds(k * 16, 16)] = ybuf[c, pl.ds(k * 16, 16)] * wv
                return 0
            lax.fori_loop(0, _C1, _srow, 0)
            pltpu.sync_copy(ybuf, accsh.at[dsti], add=True)
            return 0
        lax.fori_loop(0, _NCH1, _chunk, 0)
        plsc.subcore_barrier()
        for tt in range(_NZT):
            ch = sid + _NS * tt
            @pl.when(ch < _NZ)
            def _():
                pltpu.sync_copy(accsh.at[pl.ds(ch * _ZR, _ZR)],
                                pv_[v].at[pl.ds(cid * _N + ch * _ZR, _ZR)])
        plsc.subcore_barrier()


def _sc2_body(ya, yb, yc, se0, se1, se2, de0, de1, de2, w0, w1, w2,
              p0, p1, p2,
              ybuf, zbuf, srci, dsti, wbuf, accsh, sy):
    cid = lax.axis_index("c")
    sid = lax.axis_index("s")
    wid = sid * _NC + cid
    zv = jnp.zeros((16,), jnp.float32)

    def _zrow(r, _):
        for k in range(_D // 16):
            zbuf[r, pl.ds(k * 16, 16)] = zv
        return 0
    lax.fori_loop(0, _ZR, _zrow, 0)

    base0 = wid * _EPW
    yv_ = (ya, yb, yc)
    sev_ = (se0, se1, se2)
    dev_ = (de0, de1, de2)
    wv_ = (w0, w1, w2)
    pv_ = (p0, p1, p2)

    for v in range(_V):
        for tt in range(_NZT):
            ch = sid + _NS * tt
            @pl.when(ch < _NZ)
            def _():
                pltpu.sync_copy(zbuf, accsh.at[pl.ds(ch * _ZR, _ZR)])
        plsc.subcore_barrier()

        def _chunk(i, _, v=v):
            base = base0 + i * _C2
            pltpu.sync_copy(sev_[v].at[pl.ds(base, _C2)], srci)
            pltpu.sync_copy(dev_[v].at[pl.ds(base, _C2)], dsti)
            pltpu.sync_copy(wv_[v].at[pl.ds(base, _C2)], wbuf.at[pl.ds(0, _C2)])
            cy = pltpu.async_copy(yv_[v].at[srci], ybuf, sy)
            cy.wait()

            def _srow(c, _):
                wv = jnp.full((16,), wbuf[pl.ds(c, 16)][0])
                for k in range(_D // 16):
                    ybuf[c, pl.ds(k * 16, 16)] = ybuf[c, pl.ds(k * 16, 16)] * wv
                return 0
            lax.fori_loop(0, _C2, _srow, 0)
            pltpu.sync_copy(ybuf, accsh.at[dsti], add=True)
            return 0
        lax.fori_loop(0, _NCH2, _chunk, 0)
        plsc.subcore_barrier()
        for tt in range(_NZT):
            ch = sid + _NS * tt
            @pl.when(ch < _NZ)
            def _():
                pltpu.sync_copy(accsh.at[pl.ds(ch * _ZR, _ZR)],
                                pv_[v].at[pl.ds(cid * _N + ch * _ZR, _ZR)])
        plsc.subcore_barrier()


def _full(i):
    return (0,) * i


def kernel(x_proj, edge_indices, edge_scores, cancer_type_id, causal_scores,
           cancer_table, rw_W1_0, rw_b1_0, rw_W2_0, rw_b2_0, lin1_0, lin2_0,
           rw_W1_1, rw_b1_1, rw_W2_1, rw_b2_1, lin1_1, lin2_1,
           rw_W1_2, rw_b1_2, rw_W2_2, rw_b2_2, lin1_2, lin2_2):
    f32 = jnp.float32
    W1 = jnp.stack([rw_W1_0, rw_W1_1, rw_W1_2])
    W1 = jnp.pad(W1, ((0, 0), (0, 0), (0, _HP - _HID)))
    b1 = jnp.pad(jnp.stack([rw_b1_0, rw_b1_1, rw_b1_2]),
                 ((0, 0), (0, _HP - _HID)))[:, None, :]
    W2 = jnp.pad(jnp.stack([rw_W2_0, rw_W2_1, rw_W2_2])[..., 0],
                 ((0, 0), (0, _HP - _HID)))
    b2 = jnp.broadcast_to(jnp.stack([rw_b2_0, rw_b2_1, rw_b2_2]), (_V, 16))
    w1a = W1[:, :_D]
    w1b = W1[:, _D:2 * _D]
    w1c = W1[:, 2 * _D:2 * _D + 32]
    uvec = (W1[:, 2 * _D + 32] + W1[:, 2 * _D + 34])[:, None, :]
    tvec = (W1[:, 2 * _D + 33] - W1[:, 2 * _D + 34])[:, None, :]
    lin1 = jnp.stack([lin1_0, lin1_1, lin1_2])
    lin2 = jnp.stack([lin2_0, lin2_1, lin2_2])
    cid = cancer_type_id.astype(jnp.int32)

    pad = _EP - _E
    se = jnp.pad(edge_indices[:, 0, :], ((0, 0), (0, pad)))
    de = jnp.pad(edge_indices[:, 1, :], ((0, 0), (0, pad)))
    sc = jnp.pad(edge_scores, ((0, 0), (0, pad)))

    full = lambda shape: pl.BlockSpec(shape, lambda i: _full(len(shape)))
    tc1 = pl.pallas_call(
        _tc1_body,
        grid=(_GB,),
        in_specs=[
            pl.BlockSpec((_BR, _D), lambda i: (i, 0)),
            pl.BlockSpec((_BR, 1), lambda i: (i, 0)),
            full((16, 32)),
            pl.BlockSpec(memory_space=pltpu.SMEM),
            full((_V, _D, _HP)),
            full((_V, _D, _HP)),
            full((_V, 32, _HP)),
            full((_V, 1, _HP)),
            full((_V, 1, _HP)),
            full((_V, 1, _HP)),
            full((_V, _D, _D)),
        ],
        out_specs=[
            pl.BlockSpec((_V, _BR, _HP), lambda i: (0, i, 0)),
            pl.BlockSpec((_V, _BR, _HP), lambda i: (0, i, 0)),
            pl.BlockSpec((_V, _BR, _D), lambda i: (0, i, 0)),
            pl.BlockSpec((_BR, _D), lambda i: (i, 0)),
        ],
        out_shape=[
            jax.ShapeDtypeStruct((_V, _N, _HP), f32),
            jax.ShapeDtypeStruct((_V, _N, _HP), f32),
            jax.ShapeDtypeStruct((_V, _N, _D), f32),
            jax.ShapeDtypeStruct((_N, _D), f32),
        ],
    )
    ps, pd, y1, xh = tc1(x_proj, causal_scores, cancer_table, cid,
                         w1a, w1b, w1c, uvec, tvec, b1, lin1)

    mesh = plsc.VectorSubcoreMesh(core_axis_name="c", subcore_axis_name="s")
    sc_params = pltpu.CompilerParams(needs_layout_passes=False)
    sc1 = pl.kernel(
        _sc1_body,
        compiler_params=sc_params,
        out_type=[jax.ShapeDtypeStruct((_EP,), f32)] * 3
                 + [jax.ShapeDtypeStruct((2 * _N, _D), f32)] * 3,
        mesh=mesh,
        scratch_types=[
            pltpu.VMEM((_C1, _HP), f32),
            pltpu.VMEM((_C1, _HP), f32),
            pltpu.VMEM((_C1, _D), f32),
            pltpu.VMEM((_ZR, _D), f32),
            pltpu.VMEM((_C1,), jnp.int32),
            pltpu.VMEM((_C1,), jnp.int32),
            pltpu.VMEM((_C1,), f32),
            pltpu.VMEM((_C1 + 16,), f32),
            pltpu.VMEM((_HP,), f32),
            pltpu.VMEM((16,), f32),
            pltpu.VMEM_SHARED((_N, _D), f32),
            pltpu.SemaphoreType.DMA,
            pltpu.SemaphoreType.DMA,
            pltpu.SemaphoreType.DMA,
        ],
    )
    w0, w1_, w2_, q0, q1, q2 = sc1(
        ps[0], ps[1], ps[2], pd[0], pd[1], pd[2], y1[0], y1[1], y1[2],
        se[0], se[1], se[2], de[0], de[1], de[2], sc[0], sc[1], sc[2],
        W2[0], W2[1], W2[2], b2[0], b2[1], b2[2])

    half_a = pl.BlockSpec((_BR, _D), lambda i: (i, 0))
    half_b = pl.BlockSpec((_BR, _D), lambda i: (i + _GB, 0))
    tc2 = pl.pallas_call(
        _tc2_body,
        grid=(_GB,),
        in_specs=[half_a, half_b, half_a, half_b, half_a, half_b,
                  pl.BlockSpec((_V, _BR, _D), lambda i: (0, i, 0)),
                  pl.BlockSpec((_BR, _D), lambda i: (i, 0)),
                  full((_V, _D, _D))],
        out_specs=[pl.BlockSpec((_V, _BR, _D), lambda i: (0, i, 0)),
                   pl.BlockSpec((_V, _BR, _D), lambda i: (0, i, 0))],
        out_shape=[jax.ShapeDtypeStruct((_V, _N, _D), f32),
                   jax.ShapeDtypeStruct((_V, _N, _D), f32)],
    )
    h1, y2 = tc2(q0, q0, q1, q1, q2, q2, y1, xh, lin2)

    sc2 = pl.kernel(
        _sc2_body,
        compiler_params=sc_params,
        out_type=[jax.ShapeDtypeStruct((2 * _N, _D), f32)] * 3,
        mesh=mesh,
        scratch_types=[
            pltpu.VMEM((_C2, _D), f32),
            pltpu.VMEM((_ZR, _D), f32),
            pltpu.VMEM((_C2,), jnp.int32),
            pltpu.VMEM((_C2,), jnp.int32),
            pltpu.VMEM((_C2 + 16,), f32),
            pltpu.VMEM_SHARED((_N, _D), f32),
            pltpu.SemaphoreType.DMA,
        ],
    )
    r0, r1, r2 = sc2(y2[0], y2[1], y2[2],
                     se[0], se[1], se[2], de[0], de[1], de[2],
                     w0, w1_, w2_)

    tc3 = pl.pallas_call(
        _tc3_body,
        grid=(_GB,),
        in_specs=[half_a, half_b, half_a, half_b, half_a, half_b,
                  pl.BlockSpec((_V, _BR, _D), lambda i: (0, i, 0)),
                  pl.BlockSpec((_V, _BR, _D), lambda i: (0, i, 0))],
        out_specs=[pl.BlockSpec((_BR, _V * _D), lambda i: (i, 0)),
                   pl.BlockSpec((_V, _BR, _D), lambda i: (0, i, 0))],
        out_shape=[jax.ShapeDtypeStruct((_N, _V * _D), f32),
                   jax.ShapeDtypeStruct((_V, _N, _D), f32)],
    )
    out, h2 = tc3(r0, r0, r1, r1, r2, r2, y2, h1)
    return (out, h2[0], h2[1], h2[2])


# speedup vs baseline: 3.4403x; 1.1782x over previous
import jax
import jax.numpy as jnp
from jax import lax
from jax.experimental import pallas as pl
from jax.experimental.pallas import tpu as pltpu
from jax.experimental.pallas import tpu_sc as plsc

_N = 10000
_E = 320000
_D = 128
_HID = 145
_HP = 256
_V = 3
_NC = 2
_NS = 16
_NW = _NC * _NS
_EPW = 10240
_EP = _NW * _EPW
_CR = 32
_CS = 64
_S = 4
_NCHR = _EPW // _CR
_NCHS = _EPW // _CS
_BR = 1000
_GB = _N // _BR
_ZR = 40
_NZ = _N // _ZR
_NZT = -(-_NZ // _NS)
_EPS = 1e-15


def _tc_norm(x):
    return jnp.clip(jnp.sqrt(jnp.sum(x * x, axis=-1, keepdims=True)), _EPS, None)


def _tc_expmap0(u):
    n = _tc_norm(u)
    return jnp.tanh(n) * u / n


def _tc_logmap0(x):
    n = jnp.clip(_tc_norm(x), _EPS, 1.0 - 1e-5)
    return 0.5 * jnp.log((1.0 + n) / (1.0 - n)) * x / n


def _tc_mobius_add(x, y):
    x2 = jnp.sum(x * x, axis=-1, keepdims=True)
    y2 = jnp.sum(y * y, axis=-1, keepdims=True)
    xy = jnp.sum(x * y, axis=-1, keepdims=True)
    num = (1.0 + 2.0 * xy + y2) * x + (1.0 - x2) * y
    den = jnp.clip(1.0 + 2.0 * xy + x2 * y2, _EPS, None)
    return num / den


def _leaky(x):
    return jnp.where(x >= 0, x, 0.1 * x)


def _tc1_body(x_ref, ca_ref, tab_ref, cid_ref,
              w1a_ref, w1b_ref, w1c_ref, u_ref, t_ref, b1_ref, lin1_ref,
              ps_ref, pd_ref, y1_ref, xh_ref):
    xb = x_ref[...]
    ca = ca_ref[...]
    idx = cid_ref[0]
    onehot = (lax.broadcasted_iota(jnp.int32, (16, 1), 0) == idx
              ).astype(jnp.float32)
    ctx = jnp.sum(onehot * tab_ref[...], axis=0, keepdims=True)
    xh = _tc_expmap0(xb)
    xh_ref[...] = xh
    xtan = _tc_logmap0(xh)
    for v in range(_V):
        ps_ref[v] = (jnp.dot(xb, w1a_ref[v], preferred_element_type=jnp.float32)
                     + ca * u_ref[v])
        addv = (jnp.dot(ctx, w1c_ref[v], preferred_element_type=jnp.float32)
                + b1_ref[v])
        pd_ref[v] = (jnp.dot(xb, w1b_ref[v], preferred_element_type=jnp.float32)
                     + ca * t_ref[v] + addv)
        z = jnp.dot(xtan, lin1_ref[v], preferred_element_type=jnp.float32)
        y1_ref[v] = _tc_logmap0(_tc_expmap0(z))


def _tc2_body(pa0, pb0, pa1, pb1, pa2, pb2, y1_ref, xh_ref, lin2_ref,
              h1_ref, y2_ref):
    xh = xh_ref[...]
    pa = (pa0, pa1, pa2)
    pb = (pb0, pb1, pb2)
    for v in range(_V):
        agg = pa[v][...] + pb[v][...] + y1_ref[v]
        h = _tc_expmap0(_leaky(agg))
        h1 = _tc_mobius_add(h, xh)
        h1_ref[v] = h1
        z = jnp.dot(_tc_logmap0(h1), lin2_ref[v],
                    preferred_element_type=jnp.float32)
        y2_ref[v] = _tc_logmap0(_tc_expmap0(z))


def _tc3_body(pa0, pb0, pa1, pb1, pa2, pb2, y2_ref, h1_ref,
              out_ref, h2_ref):
    pa = (pa0, pa1, pa2)
    pb = (pb0, pb1, pb2)
    for v in range(_V):
        agg = pa[v][...] + pb[v][...] + y2_ref[v]
        h = _tc_expmap0(_leaky(agg))
        h2 = _tc_mobius_add(h, h1_ref[v])
        h2_ref[v] = h2
        out_ref[:, v * _D:(v + 1) * _D] = _tc_logmap0(h2)


def _rw_body(*refs):
    (ps0, ps1, ps2, pd0, pd1, pd2,
     se0, se1, se2, de0, de1, de2, sc0, sc1, sc2,
     w2p0, w2p1, w2p2, b2p0, b2p1, b2p2,
     wo0, wo1, wo2) = refs[:24]
    r = list(refs[24:])
    sbuf = r[0:4]; dbuf = r[4:8]; srci = r[8:12]; dsti = r[12:16]
    scb = r[16:20]; wbuf = r[20:24]; w2b = r[24]; b2b = r[25]
    sems = r[26:]
    ss = sems[0:4]; sd = sems[4:8]; isr = sems[8:12]; idd = sems[12:16]
    isc = sems[16:20]; ws = sems[20:24]

    cid = lax.axis_index("c")
    sid = lax.axis_index("s")
    wid = sid * _NC + cid
    base0 = wid * _EPW
    psv_ = (ps0, ps1, ps2)
    pdv_ = (pd0, pd1, pd2)
    sev_ = (se0, se1, se2)
    dev_ = (de0, de1, de2)
    scv_ = (sc0, sc1, sc2)
    w2v_ = (w2p0, w2p1, w2p2)
    b2v_ = (b2p0, b2p1, b2p2)
    wov_ = (wo0, wo1, wo2)
    eg = [lax.iota(jnp.int32, 16) + 16 * g for g in range(_CR // 16)]
    d_idx = sev_[0].at[pl.ds(0, _CR)]
    d_f = scv_[0].at[pl.ds(0, _CR)]
    d_row = psv_[0].at[pl.ds(0, _CR)]

    def _wait(dummy, dst, sem):
        pltpu.make_async_copy(dummy, dst, sem).wait()

    for v in range(_V):
        pltpu.sync_copy(w2v_[v], w2b)
        pltpu.sync_copy(b2v_[v], b2b)
        b2l = b2b[...]

        def _issue_idx(c, j, v=v):
            base = base0 + c * _CR
            pltpu.async_copy(sev_[v].at[pl.ds(base, _CR)], srci[j], isr[j])
            pltpu.async_copy(dev_[v].at[pl.ds(base, _CR)], dsti[j], idd[j])
            pltpu.async_copy(scv_[v].at[pl.ds(base, _CR)], scb[j], isc[j])

        def _issue_gather(j, v=v):
            pltpu.async_copy(psv_[v].at[srci[j]], sbuf[j], ss[j])
            pltpu.async_copy(pdv_[v].at[dsti[j]], dbuf[j], sd[j])

        for j in range(_S):
            _issue_idx(j, j)
        for j in range(2):
            _wait(d_idx, srci[j], isr[j])
            _wait(d_idx, dsti[j], idd[j])
            _issue_gather(j)

        def _group(g, _, v=v):
            for j in range(_S):
                c = g * _S + j
                _wait(d_row, sbuf[j], ss[j])
                _wait(d_row, dbuf[j], sd[j])
                _wait(d_f, scb[j], isc[j])
                @pl.when(c >= _S)
                def _():
                    _wait(d_f, wbuf[j], ws[j])

                def _col(col, accs):
                    colv = jnp.full((16,), col, jnp.int32)
                    w2c = jnp.full((16,), w2b[pl.ds(col, 16)][0])
                    o = []
                    for gg in range(_CR // 16):
                        sg = plsc.load_gather(sbuf[j], [eg[gg], colv])
                        dg = plsc.load_gather(dbuf[j], [eg[gg], colv])
                        o.append(accs[gg] + jnp.maximum(sg + dg, 0.0) * w2c)
                    return tuple(o)
                accs = lax.fori_loop(
                    0, _HID, _col,
                    tuple(jnp.zeros((16,), jnp.float32)
                          for _ in range(_CR // 16)))
                for gg in range(_CR // 16):
                    t = accs[gg] + b2l
                    dyn = 1.0 / (1.0 + jnp.exp(-t))
                    wbuf[j][pl.ds(gg * 16, 16)] = (
                        scb[j][pl.ds(gg * 16, 16)] * dyn)
                base = base0 + c * _CR
                pltpu.async_copy(wbuf[j], wov_[v].at[pl.ds(base, _CR)], ws[j])
                j2 = (j + 2) % _S
                @pl.when(c + 2 < _NCHR)
                def _():
                    _wait(d_idx, srci[j2], isr[j2])
                    _wait(d_idx, dsti[j2], idd[j2])
                    _issue_gather(j2)
                @pl.when(c + _S < _NCHR)
                def _():
                    _issue_idx(c + _S, j)
            return 0
        lax.fori_loop(0, _NCHR // _S, _group, 0)
        for j in range(_S):
            _wait(d_f, wbuf[j], ws[j])


def _scat_body(*refs):
    (ya, yb, yc, se0, se1, se2, de0, de1, de2, w0, w1, w2,
     p0, p1, p2) = refs[:15]
    r = list(refs[15:])
    ybuf = r[0:4]; srci = r[4:8]; dsti = r[8:12]; wbuf = r[12:16]
    accsh = r[16]
    sems = r[17:]
    gy = sems[0:4]; isr = sems[4:8]; idd = sems[8:12]; iww = sems[12:16]
    scs = sems[16:20]; zs = sems[20]

    cid = lax.axis_index("c")
    sid = lax.axis_index("s")
    wid = sid * _NC + cid
    base0 = wid * _EPW
    yv_ = (ya, yb, yc)
    sev_ = (se0, se1, se2)
    dev_ = (de0, de1, de2)
    wv_ = (w0, w1, w2)
    pv_ = (p0, p1, p2)
    zv = jnp.zeros((16,), jnp.float32)
    d_idx = sev_[0].at[pl.ds(0, _CS)]
    d_w = wv_[0].at[pl.ds(0, _CS)]
    d_row = yv_[0].at[pl.ds(0, _CS)]
    d_z = yv_[0].at[pl.ds(0, _ZR)]

    def _wait(dummy, dst, sem):
        pltpu.make_async_copy(dummy, dst, sem).wait()

    for v in range(_V):
        def _zrow(rr, _):
            for k in range(_D // 16):
                ybuf[0][rr, pl.ds(k * 16, 16)] = zv
            return 0
        lax.fori_loop(0, _ZR, _zrow, 0)
        for tt in range(_NZT):
            ch = sid + _NS * tt
            @pl.when(ch < _NZ)
            def _():
                pltpu.async_copy(ybuf[0].at[pl.ds(0, _ZR)],
                                 accsh.at[pl.ds(ch * _ZR, _ZR)], zs)
        for tt in range(_NZT):
            ch = sid + _NS * tt
            @pl.when(ch < _NZ)
            def _():
                _wait(d_z, ybuf[0].at[pl.ds(0, _ZR)], zs)
        plsc.subcore_barrier()

        def _issue_idx(c, j, v=v):
            base = base0 + c * _CS
            pltpu.async_copy(sev_[v].at[pl.ds(base, _CS)], srci[j], isr[j])
            pltpu.async_copy(dev_[v].at[pl.ds(base, _CS)], dsti[j], idd[j])
            pltpu.async_copy(wv_[v].at[pl.ds(base, _CS)],
                             wbuf[j].at[pl.ds(0, _CS)], iww[j])

        def _issue_gather(j, v=v):
            pltpu.async_copy(yv_[v].at[srci[j]], ybuf[j], gy[j])

        for j in range(_S):
            _issue_idx(j, j)
        for j in range(2):
            _wait(d_idx, srci[j], isr[j])
            _issue_gather(j)

        def _group(g, _, v=v):
            for j in range(_S):
                c = g * _S + j
                _wait(d_row, ybuf[j], gy[j])
                _wait(d_w, wbuf[j].at[pl.ds(0, _CS)], iww[j])
                _wait(d_idx, dsti[j], idd[j])

                def _srow(rr, _):
                    wv = jnp.full((16,), wbuf[j][pl.ds(rr, 16)][0])
                    for k in range(_D // 16):
                        ybuf[j][rr, pl.ds(k * 16, 16)] = (
                            ybuf[j][rr, pl.ds(k * 16, 16)] * wv)
                    return 0
                lax.fori_loop(0, _CS, _srow, 0)
                pltpu.async_copy(ybuf[j], accsh.at[dsti[j]], scs[j],
                                 add=True)
                j2 = (j + 2) % _S
                @pl.when(c >= 2)
                def _():
                    _wait(d_row, ybuf[j2], scs[j2])
                @pl.when(c + 2 < _NCHS)
                def _():
                    _wait(d_idx, srci[j2], isr[j2])
                    _issue_gather(j2)
                @pl.when(c + _S < _NCHS)
                def _():
                    _issue_idx(c + _S, j)
            return 0
        lax.fori_loop(0, _NCHS // _S, _group, 0)
        for j in ((_NCHS - 2) % _S, (_NCHS - 1) % _S):
            _wait(d_row, ybuf[j], scs[j])
        plsc.subcore_barrier()
        for tt in range(_NZT):
            ch = sid + _NS * tt
            @pl.when(ch < _NZ)
            def _():
                pltpu.sync_copy(accsh.at[pl.ds(ch * _ZR, _ZR)],
                                pv_[v].at[pl.ds(cid * _N + ch * _ZR, _ZR)])
        plsc.subcore_barrier()


def _full(i):
    return (0,) * i


def kernel(x_proj, edge_indices, edge_scores, cancer_type_id, causal_scores,
           cancer_table, rw_W1_0, rw_b1_0, rw_W2_0, rw_b2_0, lin1_0, lin2_0,
           rw_W1_1, rw_b1_1, rw_W2_1, rw_b2_1, lin1_1, lin2_1,
           rw_W1_2, rw_b1_2, rw_W2_2, rw_b2_2, lin1_2, lin2_2):
    f32 = jnp.float32
    W1 = jnp.stack([rw_W1_0, rw_W1_1, rw_W1_2])
    W1 = jnp.pad(W1, ((0, 0), (0, 0), (0, _HP - _HID)))
    b1 = jnp.pad(jnp.stack([rw_b1_0, rw_b1_1, rw_b1_2]),
                 ((0, 0), (0, _HP - _HID)))[:, None, :]
    W2 = jnp.pad(jnp.stack([rw_W2_0, rw_W2_1, rw_W2_2])[..., 0],
                 ((0, 0), (0, _HP - _HID)))
    b2 = jnp.broadcast_to(jnp.stack([rw_b2_0, rw_b2_1, rw_b2_2]), (_V, 16))
    w1a = W1[:, :_D]
    w1b = W1[:, _D:2 * _D]
    w1c = W1[:, 2 * _D:2 * _D + 32]
    uvec = (W1[:, 2 * _D + 32] + W1[:, 2 * _D + 34])[:, None, :]
    tvec = (W1[:, 2 * _D + 33] - W1[:, 2 * _D + 34])[:, None, :]
    lin1 = jnp.stack([lin1_0, lin1_1, lin1_2])
    lin2 = jnp.stack([lin2_0, lin2_1, lin2_2])
    cid = cancer_type_id.astype(jnp.int32)

    pad = _EP - _E
    se = jnp.pad(edge_indices[:, 0, :], ((0, 0), (0, pad)))
    de = jnp.pad(edge_indices[:, 1, :], ((0, 0), (0, pad)))
    sc = jnp.pad(edge_scores, ((0, 0), (0, pad)))

    full = lambda shape: pl.BlockSpec(shape, lambda i: _full(len(shape)))
    tc1 = pl.pallas_call(
        _tc1_body,
        grid=(_GB,),
        in_specs=[
            pl.BlockSpec((_BR, _D), lambda i: (i, 0)),
            pl.BlockSpec((_BR, 1), lambda i: (i, 0)),
            full((16, 32)),
            pl.BlockSpec(memory_space=pltpu.SMEM),
            full((_V, _D, _HP)),
            full((_V, _D, _HP)),
            full((_V, 32, _HP)),
            full((_V, 1, _HP)),
            full((_V, 1, _HP)),
            full((_V, 1, _HP)),
            full((_V, _D, _D)),
        ],
        out_specs=[
            pl.BlockSpec((_V, _BR, _HP), lambda i: (0, i, 0)),
            pl.BlockSpec((_V, _BR, _HP), lambda i: (0, i, 0)),
            pl.BlockSpec((_V, _BR, _D), lambda i: (0, i, 0)),
            pl.BlockSpec((_BR, _D), lambda i: (i, 0)),
        ],
        out_shape=[
            jax.ShapeDtypeStruct((_V, _N, _HP), f32),
            jax.ShapeDtypeStruct((_V, _N, _HP), f32),
            jax.ShapeDtypeStruct((_V, _N, _D), f32),
            jax.ShapeDtypeStruct((_N, _D), f32),
        ],
    )
    ps, pd, y1, xh = tc1(x_proj, causal_scores, cancer_table, cid,
                         w1a, w1b, w1c, uvec, tvec, b1, lin1)

    mesh = plsc.VectorSubcoreMesh(core_axis_name="c", subcore_axis_name="s")
    sc_params = pltpu.CompilerParams(needs_layout_passes=False)
    rw = pl.kernel(
        _rw_body,
        compiler_params=sc_params,
        out_type=[jax.ShapeDtypeStruct((_EP,), f32)] * 3,
        mesh=mesh,
        scratch_types=(
            [pltpu.VMEM((_CR, _HP), f32)] * 4
            + [pltpu.VMEM((_CR, _HP), f32)] * 4
            + [pltpu.VMEM((_CR,), jnp.int32)] * 4
            + [pltpu.VMEM((_CR,), jnp.int32)] * 4
            + [pltpu.VMEM((_CR,), f32)] * 4
            + [pltpu.VMEM((_CR,), f32)] * 4
            + [pltpu.VMEM((_HP,), f32),
               pltpu.VMEM((16,), f32)]
            + [pltpu.SemaphoreType.DMA] * 24
        ),
    )
    w0, w1_, w2_ = rw(ps[0], ps[1], ps[2], pd[0], pd[1], pd[2],
                      se[0], se[1], se[2], de[0], de[1], de[2],
                      sc[0], sc[1], sc[2],
                      W2[0], W2[1], W2[2], b2[0], b2[1], b2[2])

    scat_scratch = (
        [pltpu.VMEM((_CS, _D), f32)] * 4
        + [pltpu.VMEM((_CS,), jnp.int32)] * 4
        + [pltpu.VMEM((_CS,), jnp.int32)] * 4
        + [pltpu.VMEM((_CS + 16,), f32)] * 4
        + [pltpu.VMEM_SHARED((_N, _D), f32)]
        + [pltpu.SemaphoreType.DMA] * 21
    )
    scat1 = pl.kernel(
        _scat_body,
        compiler_params=sc_params,
        out_type=[jax.ShapeDtypeStruct((2 * _N, _D), f32)] * 3,
        mesh=mesh,
        scratch_types=scat_scratch,
    )
    q0, q1, q2 = scat1(y1[0], y1[1], y1[2],
                       se[0], se[1], se[2], de[0], de[1], de[2],
                       w0, w1_, w2_)

    half_a = pl.BlockSpec((_BR, _D), lambda i: (i, 0))
    half_b = pl.BlockSpec((_BR, _D), lambda i: (i + _GB, 0))
    tc2 = pl.pallas_call(
        _tc2_body,
        grid=(_GB,),
        in_specs=[half_a, half_b, half_a, half_b, half_a, half_b,
                  pl.BlockSpec((_V, _BR, _D), lambda i: (0, i, 0)),
                  pl.BlockSpec((_BR, _D), lambda i: (i, 0)),
                  full((_V, _D, _D))],
        out_specs=[pl.BlockSpec((_V, _BR, _D), lambda i: (0, i, 0)),
                   pl.BlockSpec((_V, _BR, _D), lambda i: (0, i, 0))],
        out_shape=[jax.ShapeDtypeStruct((_V, _N, _D), f32),
                   jax.ShapeDtypeStruct((_V, _N, _D), f32)],
    )
    h1, y2 = tc2(q0, q0, q1, q1, q2, q2, y1, xh, lin2)

    scat2 = pl.kernel(
        _scat_body,
        compiler_params=sc_params,
        out_type=[jax.ShapeDtypeStruct((2 * _N, _D), f32)] * 3,
        mesh=mesh,
        scratch_types=scat_scratch,
    )
    r0, r1, r2 = scat2(y2[0], y2[1], y2[2],
                       se[0], se[1], se[2], de[0], de[1], de[2],
                       w0, w1_, w2_)

    tc3 = pl.pallas_call(
        _tc3_body,
        grid=(_GB,),
        in_specs=[half_a, half_b, half_a, half_b, half_a, half_b,
                  pl.BlockSpec((_V, _BR, _D), lambda i: (0, i, 0)),
                  pl.BlockSpec((_V, _BR, _D), lambda i: (0, i, 0))],
        out_specs=[pl.BlockSpec((_BR, _V * _D), lambda i: (i, 0)),
                   pl.BlockSpec((_V, _BR, _D), lambda i: (0, i, 0))],
        out_shape=[jax.ShapeDtypeStruct((_N, _V * _D), f32),
                   jax.ShapeDtypeStruct((_V, _N, _D), f32)],
    )
    out, h2 = tc3(r0, r0, r1, r1, r2, r2, y2, h1)
    return (out, h2[0], h2[1], h2[2])
